# baseline (speedup 1.0000x reference)
import functools

import jax
import jax.numpy as jnp
from jax import lax
from jax.experimental import pallas as pl
from jax.experimental.pallas import tpu as pltpu
from jax.experimental.pallas import tpu_sc as plsc

_NUM_CORES = 2
_NUM_SUBCORES = 16
_NUM_WORKERS = _NUM_CORES * _NUM_SUBCORES
_LANES = 16

_VPAD = 24
_DPAD = 128
_BBLK = 128
_STRIDE = 129


def _project_body(emb_ref, w_ref, b_ref, out_ref):
    out_ref[...] = lax.dot_general(
        emb_ref[...], w_ref[...],
        dimension_numbers=(((1,), (1,)), ((), ())),
        preferred_element_type=jnp.float32,
    ) + b_ref[...]


def _project_table(embed_table, proj_w, proj_b):
    emb = jnp.pad(embed_table, ((0, _VPAD - embed_table.shape[0]), (0, 0)))
    w = jnp.pad(proj_w, ((0, _DPAD - proj_w.shape[0]), (0, 0)))
    b = jnp.pad(proj_b, (0, _DPAD - proj_b.shape[0])).reshape(1, _DPAD)
    return pl.pallas_call(
        _project_body,
        out_shape=jax.ShapeDtypeStruct((_VPAD, _DPAD), jnp.float32),
    )(emb, w, b)


def _sc_lookup(table_proj, x_flat, bsz, seq, d):
    rows_per_worker = bsz // _NUM_WORKERS
    n_blocks = rows_per_worker // _BBLK
    xblk_words = _BBLK * seq
    half_seq = seq // 2

    mesh = plsc.VectorSubcoreMesh(
        core_axis_name="c", subcore_axis_name="s")

    @functools.partial(
        pl.kernel,
        out_type=jax.ShapeDtypeStruct((seq, d, bsz), jnp.float32),
        mesh=mesh,
        scratch_types=[
            pltpu.VMEM((_VPAD, _DPAD), jnp.float32),
            pltpu.VMEM((_VPAD * _STRIDE,), jnp.float32),
            pltpu.VMEM((xblk_words,), jnp.int32),
            pltpu.VMEM((2, d, _BBLK), jnp.float32),
            [pltpu.SemaphoreType.DMA] * 2,
        ],
        compiler_params=pltpu.CompilerParams(
            use_tc_tiling_on_sc=True, needs_layout_passes=False),
    )
    def lookup(table_hbm, x_hbm, out_hbm, table_v, table_f, xv, obuf, osem):
        wid = lax.axis_index("s") * _NUM_CORES + lax.axis_index("c")
        b0 = wid * rows_per_worker

        pltpu.sync_copy(table_hbm, table_v)
        lane = lax.broadcasted_iota(jnp.int32, (_LANES,), 0)
        for v in range(_VPAD):
            for c in range(d // _LANES):
                vals = table_v[v, pl.ds(c * _LANES, _LANES)]
                plsc.store_scatter(
                    table_f, [v * _STRIDE + c * _LANES + lane], vals)

        lane_off = lane * seq

        def write(l, p, bchunk):
            return pltpu.make_async_copy(
                obuf.at[p], out_hbm.at[l, :, pl.ds(bchunk, _BBLK)], osem[p])

        def build(l, p):
            bases = [
                plsc.load_gather(xv, [lane_off + (l + j * _LANES * seq)])
                * _STRIDE
                for j in range(_BBLK // _LANES)
            ]

            @plsc.parallel_loop(0, d, unroll=32)
            def _(dd):
                col = jnp.broadcast_to(dd, (_LANES,)).astype(jnp.int32)
                for j in range(_BBLK // _LANES):
                    vals = plsc.load_gather(table_f, [bases[j] + col])
                    obuf[p, dd, pl.ds(j * _LANES, _LANES)] = vals

        def body_block(bb):
            bchunk = pl.multiple_of(b0 + bb * _BBLK, _BBLK)
            pltpu.sync_copy(
                x_hbm.at[pl.ds(pl.multiple_of(bchunk * seq, 8),
                               xblk_words)], xv)

            def body(g, carry):
                for u in (0, 1):
                    l = 2 * g + u
                    p = u
                    @pl.when(jnp.logical_or(g >= 1, bb > 0))
                    def _():
                        write(l, p, bchunk).wait()
                    build(l, p)
                    write(l, p, bchunk).start()
                return carry

            lax.fori_loop(0, half_seq, body, 0)

        def blocks(bb, carry):
            body_block(bb)
            return carry

        lax.fori_loop(0, n_blocks, blocks, 0)

        lastchunk = pl.multiple_of(b0 + (n_blocks - 1) * _BBLK, _BBLK)
        write(seq - 2, 0, lastchunk).wait()
        write(seq - 1, 1, lastchunk).wait()

    return lookup(table_proj, x_flat)


def kernel(x, embed_table, proj_w, proj_b):
    bsz, seq = x.shape
    d = embed_table.shape[1]
    table_proj = _project_table(embed_table, proj_w, proj_b)
    x_flat = x.reshape(-1).astype(jnp.int32)
    out_t = _sc_lookup(table_proj, x_flat, bsz, seq, d)
    return jnp.transpose(out_t, (2, 0, 1))


# speedup vs baseline: 1.0328x; 1.0328x over previous
import functools

import jax
import jax.numpy as jnp
from jax import lax
from jax.experimental import pallas as pl
from jax.experimental.pallas import tpu as pltpu
from jax.experimental.pallas import tpu_sc as plsc

_NUM_CORES = 2
_NUM_SUBCORES = 16
_NUM_WORKERS = _NUM_CORES * _NUM_SUBCORES
_LANES = 16

_VPAD = 24
_DPAD = 128
_BBLK = 256
_STRIDE = 129


def _project_body(emb_ref, w_ref, b_ref, out_ref):
    out_ref[...] = lax.dot_general(
        emb_ref[...], w_ref[...],
        dimension_numbers=(((1,), (1,)), ((), ())),
        preferred_element_type=jnp.float32,
    ) + b_ref[...]


def _project_table(embed_table, proj_w, proj_b):
    emb = jnp.pad(embed_table, ((0, _VPAD - embed_table.shape[0]), (0, 0)))
    w = jnp.pad(proj_w, ((0, _DPAD - proj_w.shape[0]), (0, 0)))
    b = jnp.pad(proj_b, (0, _DPAD - proj_b.shape[0])).reshape(1, _DPAD)
    return pl.pallas_call(
        _project_body,
        out_shape=jax.ShapeDtypeStruct((_VPAD, _DPAD), jnp.float32),
    )(emb, w, b)


def _sc_lookup(table_proj, x_flat, bsz, seq, d):
    rows_per_worker = bsz // _NUM_WORKERS
    n_blocks = rows_per_worker // _BBLK
    xblk_words = _BBLK * seq
    half_seq = seq // 2

    mesh = plsc.VectorSubcoreMesh(
        core_axis_name="c", subcore_axis_name="s")

    @functools.partial(
        pl.kernel,
        out_type=jax.ShapeDtypeStruct((seq, d, bsz), jnp.float32),
        mesh=mesh,
        scratch_types=[
            pltpu.VMEM((_VPAD, _DPAD), jnp.float32),
            pltpu.VMEM((_VPAD * _STRIDE,), jnp.float32),
            pltpu.VMEM((xblk_words,), jnp.int32),
            pltpu.VMEM((2, d, _BBLK), jnp.float32),
            [pltpu.SemaphoreType.DMA] * 2,
        ],
        compiler_params=pltpu.CompilerParams(
            use_tc_tiling_on_sc=True, needs_layout_passes=False),
    )
    def lookup(table_hbm, x_hbm, out_hbm, table_v, table_f, xv, obuf, osem):
        wid = lax.axis_index("s") * _NUM_CORES + lax.axis_index("c")
        b0 = wid * rows_per_worker

        pltpu.sync_copy(table_hbm, table_v)
        lane = lax.broadcasted_iota(jnp.int32, (_LANES,), 0)
        for v in range(_VPAD):
            for c in range(d // _LANES):
                vals = table_v[v, pl.ds(c * _LANES, _LANES)]
                plsc.store_scatter(
                    table_f, [v * _STRIDE + c * _LANES + lane], vals)

        lane_off = lane * seq

        def write(l, p, bchunk):
            return pltpu.make_async_copy(
                obuf.at[p], out_hbm.at[l, :, pl.ds(bchunk, _BBLK)], osem[p])

        def build(l, p):
            bases = [
                plsc.load_gather(xv, [lane_off + (l + j * _LANES * seq)])
                * _STRIDE
                for j in range(_BBLK // _LANES)
            ]

            @plsc.parallel_loop(0, d, unroll=16)
            def _(dd):
                col = jnp.broadcast_to(dd, (_LANES,)).astype(jnp.int32)
                for j in range(_BBLK // _LANES):
                    vals = plsc.load_gather(table_f, [bases[j] + col])
                    obuf[p, dd, pl.ds(j * _LANES, _LANES)] = vals

        def body_block(bb):
            bchunk = pl.multiple_of(b0 + bb * _BBLK, _BBLK)
            pltpu.sync_copy(
                x_hbm.at[pl.ds(pl.multiple_of(bchunk * seq, 8),
                               xblk_words)], xv)

            def body(g, carry):
                for u in (0, 1):
                    l = 2 * g + u
                    p = u
                    @pl.when(jnp.logical_or(g >= 1, bb > 0))
                    def _():
                        write(l, p, bchunk).wait()
                    build(l, p)
                    write(l, p, bchunk).start()
                return carry

            lax.fori_loop(0, half_seq, body, 0)

        def blocks(bb, carry):
            body_block(bb)
            return carry

        lax.fori_loop(0, n_blocks, blocks, 0)

        lastchunk = pl.multiple_of(b0 + (n_blocks - 1) * _BBLK, _BBLK)
        write(seq - 2, 0, lastchunk).wait()
        write(seq - 1, 1, lastchunk).wait()

    return lookup(table_proj, x_flat)


def kernel(x, embed_table, proj_w, proj_b):
    bsz, seq = x.shape
    d = embed_table.shape[1]
    table_proj = _project_table(embed_table, proj_w, proj_b)
    x_flat = x.reshape(-1).astype(jnp.int32)
    out_t = _sc_lookup(table_proj, x_flat, bsz, seq, d)
    return jnp.transpose(out_t, (2, 0, 1))


# speedup vs baseline: 1.0845x; 1.0500x over previous
import functools

import jax
import jax.numpy as jnp
from jax import lax
from jax.experimental import pallas as pl
from jax.experimental.pallas import tpu as pltpu
from jax.experimental.pallas import tpu_sc as plsc

_NUM_CORES = 2
_NUM_SUBCORES = 16
_NUM_WORKERS = _NUM_CORES * _NUM_SUBCORES
_LANES = 16

_VPAD = 24
_DPAD = 128
_BBLK = 128
_STRIDE = 129


def _project_body(emb_ref, w_ref, b_ref, out_ref):
    out_ref[...] = lax.dot_general(
        emb_ref[...], w_ref[...],
        dimension_numbers=(((1,), (1,)), ((), ())),
        preferred_element_type=jnp.float32,
    ) + b_ref[...]


def _project_table(embed_table, proj_w, proj_b):
    emb = jnp.pad(embed_table, ((0, _VPAD - embed_table.shape[0]), (0, 0)))
    w = jnp.pad(proj_w, ((0, _DPAD - proj_w.shape[0]), (0, 0)))
    b = jnp.pad(proj_b, (0, _DPAD - proj_b.shape[0])).reshape(1, _DPAD)
    return pl.pallas_call(
        _project_body,
        out_shape=jax.ShapeDtypeStruct((_VPAD, _DPAD), jnp.float32),
    )(emb, w, b)


def _sc_lookup(table_proj, x_flat, bsz, seq, d):
    rows_per_worker = bsz // _NUM_WORKERS
    n_blocks = rows_per_worker // _BBLK
    xblk_words = _BBLK * seq
    half_seq = seq // 2

    mesh = plsc.VectorSubcoreMesh(
        core_axis_name="c", subcore_axis_name="s")

    @functools.partial(
        pl.kernel,
        out_type=jax.ShapeDtypeStruct((seq, d, bsz), jnp.float32),
        mesh=mesh,
        scratch_types=[
            pltpu.VMEM((_VPAD, _DPAD), jnp.float32),
            pltpu.VMEM((_VPAD * _STRIDE,), jnp.float32),
            pltpu.VMEM((xblk_words,), jnp.int32),
            pltpu.VMEM((2, d, _BBLK), jnp.float32),
            [pltpu.SemaphoreType.DMA] * 2,
        ],
        compiler_params=pltpu.CompilerParams(
            use_tc_tiling_on_sc=True, needs_layout_passes=False),
    )
    def lookup(table_hbm, x_hbm, out_hbm, table_v, table_f, xv, obuf, osem):
        wid = lax.axis_index("s") * _NUM_CORES + lax.axis_index("c")
        b0 = wid * rows_per_worker

        pltpu.sync_copy(table_hbm, table_v)
        lane = lax.broadcasted_iota(jnp.int32, (_LANES,), 0)
        for v in range(_VPAD):
            for c in range(d // _LANES):
                vals = table_v[v, pl.ds(c * _LANES, _LANES)]
                plsc.store_scatter(
                    table_f, [v * _STRIDE + c * _LANES + lane], vals)

        lane_off = lane * seq

        def write(l, p, bchunk):
            return pltpu.make_async_copy(
                obuf.at[p], out_hbm.at[l, :, pl.ds(bchunk, _BBLK)], osem[p])

        def build(l, p):
            bases = [
                plsc.load_gather(xv, [lane_off + (l + j * _LANES * seq)])
                * _STRIDE
                for j in range(_BBLK // _LANES)
            ]

            @plsc.parallel_loop(0, d, unroll=16)
            def _(dd):
                col = jnp.broadcast_to(dd, (_LANES,)).astype(jnp.int32)
                for j in range(_BBLK // _LANES):
                    vals = plsc.load_gather(table_f, [bases[j] + col])
                    obuf[p, dd, pl.ds(j * _LANES, _LANES)] = vals

        def body_block(bb):
            bchunk = pl.multiple_of(b0 + bb * _BBLK, _BBLK)
            pltpu.sync_copy(
                x_hbm.at[pl.ds(pl.multiple_of(bchunk * seq, 8),
                               xblk_words)], xv)

            def body(g, carry):
                for u in (0, 1):
                    l = 2 * g + u
                    p = u
                    @pl.when(jnp.logical_or(g >= 1, bb > 0))
                    def _():
                        write(l, p, bchunk).wait()
                    build(l, p)
                    write(l, p, bchunk).start()
                return carry

            lax.fori_loop(0, half_seq, body, 0)

        def blocks(bb, carry):
            body_block(bb)
            return carry

        lax.fori_loop(0, n_blocks, blocks, 0)

        lastchunk = pl.multiple_of(b0 + (n_blocks - 1) * _BBLK, _BBLK)
        write(seq - 2, 0, lastchunk).wait()
        write(seq - 1, 1, lastchunk).wait()

    return lookup(table_proj, x_flat)


def kernel(x, embed_table, proj_w, proj_b):
    bsz, seq = x.shape
    d = embed_table.shape[1]
    table_proj = _project_table(embed_table, proj_w, proj_b)
    x_flat = x.reshape(-1).astype(jnp.int32)
    out_t = _sc_lookup(table_proj, x_flat, bsz, seq, d)
    return jnp.transpose(out_t, (2, 0, 1))
